# baseline (speedup 1.0000x reference)
import jax
import jax.numpy as jnp
from jax import lax
from jax.experimental import pallas as pl
from jax.experimental.pallas import tpu as pltpu
from jax.experimental.pallas import tpu_sc as plsc

NUM_WORKERS = 32
CHUNK = 128
D = 64
L = 16


def _gather_body(xt_hbm, w_hbm, out_hbm, idx_v, gbuf_a, gbuf_b, tbuf_a,
                 tbuf_b, gsem_a, gsem_b, wsem_a, wsem_b):
  n_cols = xt_hbm.shape[0]
  c = lax.axis_index("c")
  s = lax.axis_index("s")
  wid = s * 2 + c
  pltpu.sync_copy(xt_hbm.at[:, pl.ds(wid * CHUNK, CHUNK)], idx_v)

  iot = lax.broadcasted_iota(jnp.int32, (L,), 0)

  def startg(j, gbuf, sem):
    pltpu.async_copy(w_hbm.at[idx_v.at[j]], gbuf, sem)

  def waitg(j, gbuf, sem):
    pltpu.make_async_copy(w_hbm.at[idx_v.at[j]], gbuf, sem).wait()

  def transpose(gbuf, tbuf):
    for cc in range(D):
      cvec = jnp.full((L,), cc, jnp.int32)
      for g in range(CHUNK // L):
        vals = plsc.load_gather(gbuf, [iot + g * L, cvec])
        tbuf[cc, pl.ds(g * L, L)] = vals

  def startw(j, tbuf, sem):
    for cb in range(8):
      pltpu.async_copy(tbuf.at[pl.ds(8 * cb, 8)], out_hbm.at[j, cb, wid],
                       sem)

  def waitw(j, tbuf, sem):
    for cb in range(8):
      pltpu.make_async_copy(tbuf.at[pl.ds(8 * cb, 8)],
                            out_hbm.at[j, cb, wid], sem).wait()

  startg(0, gbuf_a, gsem_a)
  startg(1, gbuf_b, gsem_b)

  def step(q, carry):
    j = 2 * q

    def half(j, gbuf, tbuf, gsem, wsem):
      waitg(j, gbuf, gsem)

      @pl.when(j >= 2)
      def _():
        waitw(j - 2, tbuf, wsem)

      transpose(gbuf, tbuf)
      startw(j, tbuf, wsem)

      @pl.when(j + 2 < n_cols)
      def _():
        startg(j + 2, gbuf, gsem)

    half(j, gbuf_a, tbuf_a, gsem_a, wsem_a)
    half(j + 1, gbuf_b, tbuf_b, gsem_b, wsem_b)
    return carry

  lax.fori_loop(0, n_cols // 2, step, 0)

  waitw(n_cols - 2, tbuf_a, wsem_a)
  waitw(n_cols - 1, tbuf_b, wsem_b)


def kernel(x, weight):
  n_emb = weight.shape[0]
  n_rows, n_cols = x.shape
  n_blk = n_rows // CHUNK

  wp = jnp.pad(weight, ((0, 0), (0, 128 - D)))
  xt = (x.astype(jnp.int32) * 2).T

  mesh = plsc.VectorSubcoreMesh(core_axis_name="c", subcore_axis_name="s")
  run = pl.kernel(
      _gather_body,
      out_type=jax.ShapeDtypeStruct((n_cols, D // 8, n_blk, 8, CHUNK),
                                    jnp.float32),
      mesh=mesh,
      scratch_types=[
          pltpu.VMEM((n_cols, CHUNK), jnp.int32),
          pltpu.VMEM((CHUNK, D), jnp.float32),
          pltpu.VMEM((CHUNK, D), jnp.float32),
          pltpu.VMEM((D, CHUNK), jnp.float32),
          pltpu.VMEM((D, CHUNK), jnp.float32),
          pltpu.SemaphoreType.DMA,
          pltpu.SemaphoreType.DMA,
          pltpu.SemaphoreType.DMA,
          pltpu.SemaphoreType.DMA,
      ],
      compiler_params=pltpu.CompilerParams(use_tc_tiling_on_sc=False,
                                           needs_layout_passes=False),
  )
  out5 = run(xt, wp.reshape(2 * n_emb, D))
  return out5.transpose(2, 4, 0, 1, 3).reshape(n_rows, n_cols, D)


# speedup vs baseline: 1.3386x; 1.3386x over previous
import jax
import jax.numpy as jnp
from jax import lax
from jax.experimental import pallas as pl
from jax.experimental.pallas import tpu as pltpu
from jax.experimental.pallas import tpu_sc as plsc

NUM_WORKERS = 32
CHUNK = 128
D = 64
L = 16


def _gather_body(xt_hbm, w_hbm, out_hbm, idx_v, gbuf_a, gbuf_b, tbuf_a,
                 tbuf_b, gsem_a, gsem_b, wsem_a, wsem_b):
  n_cols = xt_hbm.shape[0]
  c = lax.axis_index("c")
  s = lax.axis_index("s")
  wid = s * 2 + c
  pltpu.sync_copy(xt_hbm.at[:, pl.ds(wid * CHUNK, CHUNK)], idx_v)

  iot = lax.broadcasted_iota(jnp.int32, (L,), 0)

  def startg(j, gbuf, sem):
    pltpu.async_copy(w_hbm.at[idx_v.at[j]], gbuf, sem)

  def waitg(j, gbuf, sem):
    pltpu.make_async_copy(w_hbm.at[idx_v.at[j]], gbuf, sem).wait()

  def transpose(gbuf, tbuf):
    for i in range(CHUNK):
      ivec = jnp.full((L,), i, jnp.int32)
      for u in range(D // L):
        vals = gbuf[i, pl.ds(u * L, L)]
        plsc.store_scatter(tbuf, [iot + u * L, ivec], vals)

  def startw(j, tbuf, sem):
    for cb in range(8):
      pltpu.async_copy(tbuf.at[pl.ds(8 * cb, 8), pl.ds(0, CHUNK)],
                       out_hbm.at[j, cb, wid], sem)

  def waitw(j, tbuf, sem):
    for cb in range(8):
      pltpu.make_async_copy(tbuf.at[pl.ds(8 * cb, 8), pl.ds(0, CHUNK)],
                            out_hbm.at[j, cb, wid], sem).wait()

  startg(0, gbuf_a, gsem_a)
  startg(1, gbuf_b, gsem_b)

  def step(q, carry):
    j = 2 * q

    def half(j, gbuf, tbuf, gsem, wsem):
      waitg(j, gbuf, gsem)

      @pl.when(j >= 2)
      def _():
        waitw(j - 2, tbuf, wsem)

      transpose(gbuf, tbuf)
      startw(j, tbuf, wsem)

      @pl.when(j + 2 < n_cols)
      def _():
        startg(j + 2, gbuf, gsem)

    half(j, gbuf_a, tbuf_a, gsem_a, wsem_a)
    half(j + 1, gbuf_b, tbuf_b, gsem_b, wsem_b)
    return carry

  lax.fori_loop(0, n_cols // 2, step, 0)

  waitw(n_cols - 2, tbuf_a, wsem_a)
  waitw(n_cols - 1, tbuf_b, wsem_b)


def kernel(x, weight):
  n_emb = weight.shape[0]
  n_rows, n_cols = x.shape
  n_blk = n_rows // CHUNK

  wp = jnp.pad(weight, ((0, 0), (0, 128 - D)))
  xt = (x.astype(jnp.int32) * 2).T

  mesh = plsc.VectorSubcoreMesh(core_axis_name="c", subcore_axis_name="s")
  run = pl.kernel(
      _gather_body,
      out_type=jax.ShapeDtypeStruct((n_cols, D // 8, n_blk, 8, CHUNK),
                                    jnp.float32),
      mesh=mesh,
      scratch_types=[
          pltpu.VMEM((n_cols, CHUNK), jnp.int32),
          pltpu.VMEM((CHUNK, D), jnp.float32),
          pltpu.VMEM((CHUNK, D), jnp.float32),
          pltpu.VMEM((D, 136), jnp.float32),
          pltpu.VMEM((D, 136), jnp.float32),
          pltpu.SemaphoreType.DMA,
          pltpu.SemaphoreType.DMA,
          pltpu.SemaphoreType.DMA,
          pltpu.SemaphoreType.DMA,
      ],
      compiler_params=pltpu.CompilerParams(use_tc_tiling_on_sc=False,
                                           needs_layout_passes=False),
  )
  out5 = run(xt, wp.reshape(2 * n_emb, D))
  return out5.transpose(2, 4, 0, 1, 3).reshape(n_rows, n_cols, D)
